# Initial kernel scaffold; baseline (speedup 1.0000x reference)
import functools

import jax
import jax.numpy as jnp
from jax.experimental import pallas as pl

_NUM_SAMPLES = 100
_SIGMA = 0.05
_K_FRAC = 0.01


def _topk_onehot_mean_body(x_ref, noise_ref, out_ref, *, k, t, ns):
    x = x_ref[0, :]
    noise = noise_ref[0]
    vals = x[None, :] + noise * _SIGMA
    iota = jax.lax.broadcasted_iota(jnp.int32, (ns, t), 1)
    inv_ns = 1.0 / ns
    for j in range(k):
        m = jnp.max(vals, axis=1, keepdims=True)
        am = jnp.min(jnp.where(vals == m, iota, t), axis=1, keepdims=True)
        hit = iota == am
        out_ref[0, j, :] = jnp.sum(hit.astype(jnp.float32), axis=0) * inv_ns
        vals = jnp.where(hit, -jnp.inf, vals)


def kernel(x, train_mode):
    b, t = x.shape
    k = max(1, min(int(t * _K_FRAC), t))
    del train_mode
    noise = jax.random.normal(
        jax.random.key(1), (b, _NUM_SAMPLES, t), dtype=jnp.float32
    )
    body = functools.partial(_topk_onehot_mean_body, k=k, t=t, ns=_NUM_SAMPLES)
    out = pl.pallas_call(
        body,
        grid=(b,),
        in_specs=[
            pl.BlockSpec((1, t), lambda i: (i, 0)),
            pl.BlockSpec((1, _NUM_SAMPLES, t), lambda i: (i, 0, 0)),
        ],
        out_specs=pl.BlockSpec((1, k, t), lambda i: (i, 0, 0)),
        out_shape=jax.ShapeDtypeStruct((b, k, t), jnp.float32),
    )(x, noise)
    return out


# speedup vs baseline: 7.4135x; 7.4135x over previous
import functools

import jax
import jax.numpy as jnp
from jax.experimental import pallas as pl

_NUM_SAMPLES = 100
_SIGMA = 0.05
_K_FRAC = 0.01


def _topk_onehot_mean_body(x_ref, noise_ref, out_ref, *, k, t, ns):
    x = x_ref[0, 0, :]
    noise = noise_ref[0]
    vals = x[None, :] + noise * _SIGMA
    iota = jax.lax.broadcasted_iota(jnp.int32, (ns, t), 1)
    inv_ns = 1.0 / ns
    for j in range(k):
        m = jnp.max(vals, axis=1, keepdims=True)
        am = jnp.min(jnp.where(vals == m, iota, t), axis=1, keepdims=True)
        hit = iota == am
        out_ref[0, j, :] = jnp.sum(hit.astype(jnp.float32), axis=0) * inv_ns
        vals = jnp.where(hit, -jnp.inf, vals)


def kernel(x, train_mode):
    b, t = x.shape
    k = max(1, min(int(t * _K_FRAC), t))
    del train_mode
    noise = jax.random.normal(
        jax.random.key(1), (b, _NUM_SAMPLES, t), dtype=jnp.float32
    )
    body = functools.partial(_topk_onehot_mean_body, k=k, t=t, ns=_NUM_SAMPLES)
    out = pl.pallas_call(
        body,
        grid=(b,),
        in_specs=[
            pl.BlockSpec((1, 1, t), lambda i: (i, 0, 0)),
            pl.BlockSpec((1, _NUM_SAMPLES, t), lambda i: (i, 0, 0)),
        ],
        out_specs=pl.BlockSpec((1, k, t), lambda i: (i, 0, 0)),
        out_shape=jax.ShapeDtypeStruct((b, k, t), jnp.float32),
    )(x.reshape(b, 1, t), noise)
    return out
